# baseline (speedup 1.0000x reference)
import functools

import jax
import jax.numpy as jnp
from jax import lax
from jax.experimental import pallas as pl
from jax.experimental.pallas import tpu as pltpu
from jax.experimental.pallas import tpu_sc as plsc

N = 50000
E = 800000
H = 64
BN = 2000
BE = 1600

_F32 = jnp.float32

_NC = 2
_NS = 16
_NW = _NC * _NS
_GCH = 128

_NHALF = N // _NC
_PROWS = _NHALF // 2
_DUMMY = _PROWS
_RSH = 12544
_ZR = _RSH // _NS
_SCH = 96
_OROWS = 784

_EA = 409600
_EB = E - _EA

_sc_mesh = plsc.VectorSubcoreMesh(core_axis_name="c", subcore_axis_name="s")


def _embed_body(x_ref, w_ref, b_ref, o_ref):
    o_ref[...] = (
        jnp.dot(x_ref[...], w_ref[...], preferred_element_type=_F32) + b_ref[...]
    )


def _embed(xp, Wp, b2):
    return pl.pallas_call(
        _embed_body,
        grid=(N // BN,),
        in_specs=[
            pl.BlockSpec((BN, 128), lambda i: (i, 0)),
            pl.BlockSpec((128, H), lambda i: (0, 0)),
            pl.BlockSpec((1, H), lambda i: (0, 0)),
        ],
        out_specs=pl.BlockSpec((BN, H), lambda i: (i, 0)),
        out_shape=jax.ShapeDtypeStruct((N, H), _F32),
    )(xp, Wp, b2)


def _proj_body(h_ref, wd_ref, ws_ref, pd_ref, ps_ref):
    h = h_ref[...]
    pd_ref[...] = jnp.dot(h, wd_ref[...], preferred_element_type=_F32)
    ps_ref[...] = jnp.dot(h, ws_ref[...], preferred_element_type=_F32)


def _proj(h, Wd, Ws):
    return pl.pallas_call(
        _proj_body,
        grid=(N // BN,),
        in_specs=[
            pl.BlockSpec((BN, H), lambda i: (i, 0)),
            pl.BlockSpec((H, 2 * H), lambda i: (0, 0)),
            pl.BlockSpec((H, 2 * H), lambda i: (0, 0)),
        ],
        out_specs=[
            pl.BlockSpec((BN, 2 * H), lambda i: (i, 0)),
            pl.BlockSpec((BN, 2 * H), lambda i: (i, 0)),
        ],
        out_shape=[
            jax.ShapeDtypeStruct((N, 2 * H), _F32),
            jax.ShapeDtypeStruct((N, 2 * H), _F32),
        ],
    )(h, Wd, Ws)


def _msg_body(g_ref, ea_ref, we_ref, b_ref, par_ref, o_ref):
    zf = g_ref[...] + jnp.dot(ea_ref[...], we_ref[...],
                              preferred_element_type=_F32) + b_ref[...]
    zg = zf[:, :H]
    zc = zf[:, H:]
    sig = 1.0 / (1.0 + jnp.exp(-zg))
    sp = jnp.maximum(zc, 0.0) + jnp.log(1.0 + jnp.exp(-jnp.abs(zc)))
    m = sig * sp
    par = par_ref[...]
    o_ref[...] = jnp.concatenate([m * (1.0 - par), m * par], axis=1)


def _make_msg(ne):
    def call(G, ea, We, b2, par):
        return pl.pallas_call(
            _msg_body,
            grid=(ne // BE,),
            in_specs=[
                pl.BlockSpec((BE, 2 * H), lambda i: (i, 0)),
                pl.BlockSpec((BE, 16), lambda i: (i, 0)),
                pl.BlockSpec((16, 2 * H), lambda i: (0, 0)),
                pl.BlockSpec((1, 2 * H), lambda i: (0, 0)),
                pl.BlockSpec((BE, 1), lambda i: (i, 0)),
            ],
            out_specs=pl.BlockSpec((BE, 2 * H), lambda i: (i, 0)),
            out_shape=jax.ShapeDtypeStruct((ne, 2 * H), _F32),
        )(G, ea, We, b2, par)
    return call


def _upd_body(h_ref, a_ref, b_ref, o_ref):
    t = h_ref[...] + a_ref[...] + b_ref[...]
    o_ref[...] = jnp.maximum(t, 0.0) + jnp.log(1.0 + jnp.exp(-jnp.abs(t)))


def _upd(h, aggA, aggB):
    return pl.pallas_call(
        _upd_body,
        grid=(N // BN,),
        in_specs=[
            pl.BlockSpec((BN, H), lambda i: (i, 0)),
            pl.BlockSpec((BN, H), lambda i: (i, 0)),
            pl.BlockSpec((BN, H), lambda i: (i, 0)),
        ],
        out_specs=pl.BlockSpec((BN, H), lambda i: (i, 0)),
        out_shape=jax.ShapeDtypeStruct((N, H), _F32),
    )(h, aggA, aggB)


def _make_gather(ne):
    epw = ne // _NW
    nfull = epw // _GCH
    grem = epw - nfull * _GCH
    npair = (nfull - 1) // 2
    k0 = 2 * npair

    ngrp = max((nfull - 5) // 3, 0)

    @functools.partial(
        pl.kernel,
        out_type=jax.ShapeDtypeStruct((ne, 2 * H), _F32),
        mesh=_sc_mesh,
        scratch_types=[
            pltpu.VMEM((epw,), jnp.int32),
            pltpu.VMEM((epw,), jnp.int32),
            pltpu.VMEM((_GCH, 2 * H), _F32),
            pltpu.VMEM((_GCH, 2 * H), _F32),
            pltpu.VMEM((_GCH, 2 * H), _F32),
            pltpu.SemaphoreType.DMA,
            pltpu.SemaphoreType.DMA,
            pltpu.SemaphoreType.DMA,
            pltpu.SemaphoreType.DMA,
            pltpu.SemaphoreType.DMA,
            pltpu.SemaphoreType.DMA,
            pltpu.SemaphoreType.DMA,
            pltpu.SemaphoreType.DMA,
            pltpu.SemaphoreType.DMA,
        ],
    )
    def gather_k(pd_hbm, ps_hbm, dst_hbm, src_hbm, out_hbm, dsti, srci,
                 buf0, buf1, buf2, semA0, semA1, semA2,
                 semB0, semB1, semB2, semS0, semS1, semS2):
        wid = lax.axis_index("s") * _NC + lax.axis_index("c")
        base = wid * epw
        pltpu.sync_copy(dst_hbm.at[pl.ds(base, epw)], dsti)
        pltpu.sync_copy(src_hbm.at[pl.ds(base, epw)], srci)

        bufs = (buf0, buf1, buf2)
        semsA = (semA0, semA1, semA2)
        semsB = (semB0, semB1, semB2)
        semsS = (semS0, semS1, semS2)

        def issue_a(off, slot):
            pltpu.async_copy(pd_hbm.at[dsti.at[pl.ds(off, _GCH)]],
                             bufs[slot], semsA[slot])

        def wait_a(off, slot):
            pltpu.make_async_copy(pd_hbm.at[dsti.at[pl.ds(off, _GCH)]],
                                  bufs[slot], semsA[slot]).wait()

        def issue_b(off, slot):
            pltpu.async_copy(ps_hbm.at[srci.at[pl.ds(off, _GCH)]],
                             bufs[slot], semsB[slot], add=True)

        def wait_b(off, slot):
            pltpu.make_async_copy(ps_hbm.at[srci.at[pl.ds(off, _GCH)]],
                                  bufs[slot], semsB[slot]).wait()

        def issue_s(off, slot):
            pltpu.async_copy(bufs[slot], out_hbm.at[pl.ds(base + off, _GCH)],
                             semsS[slot])

        def wait_s(off, slot):
            pltpu.make_async_copy(bufs[slot],
                                  out_hbm.at[pl.ds(base + off, _GCH)],
                                  semsS[slot]).wait()

        def step(k_off, u):
            if u is None:
                return
            slot = u % 3
            nslot = (u + 2) % 3
            pltpu.make_async_copy(
                bufs[nslot],
                out_hbm.at[pl.ds(base + k_off - _GCH, _GCH)],
                semsS[nslot]).wait()
            issue_a(k_off + 2 * _GCH, nslot)
            wait_a(k_off, slot)
            issue_b(k_off, slot)
            wait_b(k_off, slot)
            issue_s(k_off, slot)

        issue_a(0, 0)
        issue_a(_GCH, 1)
        issue_a(2 * _GCH, 2)
        wait_a(0, 0)
        issue_b(0, 0)
        wait_b(0, 0)
        issue_s(0, 0)

        @pl.loop(0, ngrp)
        def _(j):
            k = (3 * j + 1) * _GCH
            step(k, 1)
            step(k + _GCH, 2)
            step(k + 2 * _GCH, 0)

        issued = 3 * ngrp + 3 if ngrp else 3
        for k in range(3 * ngrp + 1, nfull):
            slot = k % 3
            nslot = (k + 2) % 3
            wait_s((k - 1) * _GCH, nslot)
            if k + 2 < nfull and k + 2 >= issued:
                issue_a((k + 2) * _GCH, nslot)
                issued = k + 3
            wait_a(k * _GCH, slot)
            issue_b(k * _GCH, slot)
            wait_b(k * _GCH, slot)
            issue_s(k * _GCH, slot)
        if grem:
            roff = nfull * _GCH
            slot = nfull % 3
            pltpu.async_copy(pd_hbm.at[dsti.at[pl.ds(roff, grem)]],
                             bufs[slot].at[pl.ds(0, grem)], semsA[slot]).wait()
            pltpu.async_copy(ps_hbm.at[srci.at[pl.ds(roff, grem)]],
                             bufs[slot].at[pl.ds(0, grem)], semsB[slot],
                             add=True).wait()
            pltpu.sync_copy(bufs[slot].at[pl.ds(0, grem)],
                            out_hbm.at[pl.ds(base + roff, grem)])
        wait_s((nfull - 1) * _GCH, (nfull - 1) % 3)

    return gather_k


def _make_scatter(ne):
    etp = ne // _NS
    sfull = etp // _SCH
    srem = etp - sfull * _SCH
    npair = (sfull - 1) // 2
    k0 = 2 * npair

    @functools.partial(
        pl.kernel,
        out_type=jax.ShapeDtypeStruct((2 * _RSH, 2 * H), _F32),
        mesh=_sc_mesh,
        scratch_types=[
            pltpu.VMEM((_SCH,), jnp.int32),
            pltpu.VMEM((_SCH,), jnp.int32),
            pltpu.VMEM((_SCH, 2 * H), _F32),
            pltpu.VMEM((_SCH, 2 * H), _F32),
            pltpu.VMEM((_SCH,), jnp.int32),
            pltpu.VMEM_SHARED((_RSH, 2 * H), _F32),
            pltpu.SemaphoreType.DMA,
            pltpu.SemaphoreType.DMA,
            pltpu.SemaphoreType.DMA,
            pltpu.SemaphoreType.DMA,
        ],
    )
    def scatter_k(msg_hbm, dst_hbm, zeros_hbm, out_hbm,
                  idx0, idx1, mbuf0, mbuf1, sidx, aggsh,
                  semM0, semM1, semI0, semI1):
        c = lax.axis_index("c")
        s = lax.axis_index("s")
        nbase = c * _NHALF
        obase = c * _RSH
        ebase = s * etp
        pltpu.sync_copy(zeros_hbm, mbuf0)
        for z in range(_ZR // _SCH):
            pltpu.sync_copy(mbuf0, aggsh.at[pl.ds(s * _ZR + z * _SCH, _SCH)])
        pltpu.sync_copy(mbuf0.at[pl.ds(0, _ZR - (_ZR // _SCH) * _SCH)],
                        aggsh.at[pl.ds(s * _ZR + (_ZR // _SCH) * _SCH,
                                       _ZR - (_ZR // _SCH) * _SCH)])
        plsc.subcore_barrier()

        mbufs = (mbuf0, mbuf1)
        idxs = (idx0, idx1)
        semsM = (semM0, semM1)
        semsI = (semI0, semI1)

        def issue_l(k, slot):
            goff = ebase + k * _SCH
            pltpu.async_copy(msg_hbm.at[pl.ds(goff, _SCH)], mbufs[slot], semsM[slot])
            pltpu.async_copy(dst_hbm.at[pl.ds(goff, _SCH)], idxs[slot], semsI[slot])

        def wait_l(k, slot):
            goff = ebase + k * _SCH
            pltpu.make_async_copy(msg_hbm.at[pl.ds(goff, _SCH)],
                                  mbufs[slot], semsM[slot]).wait()
            pltpu.make_async_copy(dst_hbm.at[pl.ds(goff, _SCH)],
                                  idxs[slot], semsI[slot]).wait()

        def finish(slot, nvec):
            for v in range(nvec):
                d = idxs[slot][pl.ds(v * 16, 16)]
                li = d - nbase
                ok = (li >= 0) & (li < _NHALF)
                sidx[pl.ds(v * 16, 16)] = jnp.where(ok, li >> 1, _DUMMY)
            for v in range(nvec, _SCH // 16):
                sidx[pl.ds(v * 16, 16)] = jnp.full((16,), _DUMMY, jnp.int32)
            pltpu.sync_copy(mbufs[slot], aggsh.at[sidx], add=True)

        issue_l(0, 0)

        @pl.loop(0, npair)
        def _(j):
            a = 2 * j
            issue_l(a + 1, 1)
            wait_l(a, 0)
            finish(0, _SCH // 16)
            issue_l(a + 2, 0)
            wait_l(a + 1, 1)
            finish(1, _SCH // 16)

        wait_l(k0, 0)
        finish(0, _SCH // 16)
        if k0 + 1 < sfull:
            issue_l(k0 + 1, 1)
            wait_l(k0 + 1, 1)
            finish(1, _SCH // 16)
        if srem:
            roff = ebase + sfull * _SCH
            pltpu.async_copy(msg_hbm.at[pl.ds(roff, srem)],
                             mbuf0.at[pl.ds(0, srem)], semM0).wait()
            pltpu.async_copy(dst_hbm.at[pl.ds(roff, srem)],
                             idx0.at[pl.ds(0, srem)], semI0).wait()
            for v in range(srem // 16):
                d = idx0[pl.ds(v * 16, 16)]
                li = d - nbase
                ok = (li >= 0) & (li < _NHALF)
                sidx[pl.ds(v * 16, 16)] = jnp.where(ok, li >> 1, _DUMMY)
            for v in range(srem // 16, _SCH // 16):
                sidx[pl.ds(v * 16, 16)] = jnp.full((16,), _DUMMY, jnp.int32)
            pltpu.sync_copy(mbuf0, aggsh.at[sidx], add=True)
        plsc.subcore_barrier()

        def out_rows(roff2, rsz):
            pltpu.sync_copy(aggsh.at[pl.ds(roff2, rsz)], mbuf0.at[pl.ds(0, rsz)])
            pltpu.sync_copy(mbuf0.at[pl.ds(0, rsz)],
                            out_hbm.at[pl.ds(obase + roff2, rsz)])

        @pl.when(s < _NS - 1)
        def _():
            for z in range(_OROWS // _SCH):
                out_rows(s * _OROWS + z * _SCH, _SCH)
            out_rows(s * _OROWS + (_OROWS // _SCH) * _SCH,
                     _OROWS - (_OROWS // _SCH) * _SCH)

        @pl.when(s == _NS - 1)
        def _():
            last = 744
            for z in range(last // _SCH):
                out_rows((_NS - 1) * _OROWS + z * _SCH, _SCH)
            out_rows((_NS - 1) * _OROWS + (last // _SCH) * _SCH,
                     last - (last // _SCH) * _SCH)

    return scatter_k


_gather_a = _make_gather(_EA)
_gather_b = _make_gather(_EB)
_scatter_a = _make_scatter(_EA)
_scatter_b = _make_scatter(_EB)
_msg_a = _make_msg(_EA)
_msg_b = _make_msg(_EB)


def kernel(x, edge_index, edge_attr, W_embed, b_embed,
           W_full_0, b_full_0, W_full_1, b_full_1, W_full_2, b_full_2):
    src = edge_index[0]
    dst = edge_index[1]
    xp = jnp.pad(x, ((0, 0), (0, 128 - x.shape[1])))
    Wp = jnp.pad(W_embed, ((0, 128 - W_embed.shape[0]), (0, 0)))
    h = _embed(xp, Wp, b_embed.reshape(1, H))
    zeros_sh = jnp.zeros((_SCH, 2 * H), _F32)
    par = (dst & 1).astype(_F32).reshape(E, 1)
    dst_a, dst_b = dst[:_EA], dst[_EA:]
    src_a, src_b = src[:_EA], src[_EA:]
    ea_a, ea_b = edge_attr[:_EA], edge_attr[_EA:]
    par_a, par_b = par[:_EA], par[_EA:]
    for W, b in ((W_full_0, b_full_0), (W_full_1, b_full_1), (W_full_2, b_full_2)):
        Wd, Ws, We = W[:H], W[H:2 * H], W[2 * H:]
        b2 = b.reshape(1, 2 * H)
        Pd, Ps = _proj(h, Wd, Ws)
        Ga = _gather_a(Pd, Ps, dst_a, src_a)
        Gb = _gather_b(Pd, Ps, dst_b, src_b)
        Ma = _msg_a(Ga, ea_a, We, b2, par_a)
        Mb = _msg_b(Gb, ea_b, We, b2, par_b)
        apA = _scatter_a(Ma, dst_a, zeros_sh)
        apB = _scatter_b(Mb, dst_b, zeros_sh)
        aggA = jnp.concatenate(
            [apA[:_PROWS], apA[_RSH:_RSH + _PROWS]], axis=0).reshape(N, H)
        aggB = jnp.concatenate(
            [apB[:_PROWS], apB[_RSH:_RSH + _PROWS]], axis=0).reshape(N, H)
        h = _upd(h, aggA, aggB)
    return h


# speedup vs baseline: 1.0657x; 1.0657x over previous
import functools

import jax
import jax.numpy as jnp
from jax import lax
from jax.experimental import pallas as pl
from jax.experimental.pallas import tpu as pltpu
from jax.experimental.pallas import tpu_sc as plsc

N = 50000
E = 800000
H = 64
BN = 2000
BE = 1600

_F32 = jnp.float32

_NC = 2
_NS = 16
_NW = _NC * _NS
_GCH = 128

_NHALF = N // _NC
_PROWS = _NHALF // 2
_DUMMY = _PROWS
_RSH = 12544
_ZR = _RSH // _NS
_SCH = 96
_OROWS = 784

_SPLITS = (268800, 268800, 262400)

_sc_mesh = plsc.VectorSubcoreMesh(core_axis_name="c", subcore_axis_name="s")


def _embed_body(x_ref, w_ref, b_ref, o_ref):
    o_ref[...] = (
        jnp.dot(x_ref[...], w_ref[...], preferred_element_type=_F32) + b_ref[...]
    )


def _embed(xp, Wp, b2):
    return pl.pallas_call(
        _embed_body,
        grid=(N // BN,),
        in_specs=[
            pl.BlockSpec((BN, 128), lambda i: (i, 0)),
            pl.BlockSpec((128, H), lambda i: (0, 0)),
            pl.BlockSpec((1, H), lambda i: (0, 0)),
        ],
        out_specs=pl.BlockSpec((BN, H), lambda i: (i, 0)),
        out_shape=jax.ShapeDtypeStruct((N, H), _F32),
    )(xp, Wp, b2)


def _proj_body(h_ref, wd_ref, ws_ref, pd_ref, ps_ref):
    h = h_ref[...]
    pd_ref[...] = jnp.dot(h, wd_ref[...], preferred_element_type=_F32)
    ps_ref[...] = jnp.dot(h, ws_ref[...], preferred_element_type=_F32)


def _proj(h, Wd, Ws):
    return pl.pallas_call(
        _proj_body,
        grid=(N // BN,),
        in_specs=[
            pl.BlockSpec((BN, H), lambda i: (i, 0)),
            pl.BlockSpec((H, 2 * H), lambda i: (0, 0)),
            pl.BlockSpec((H, 2 * H), lambda i: (0, 0)),
        ],
        out_specs=[
            pl.BlockSpec((BN, 2 * H), lambda i: (i, 0)),
            pl.BlockSpec((BN, 2 * H), lambda i: (i, 0)),
        ],
        out_shape=[
            jax.ShapeDtypeStruct((N, 2 * H), _F32),
            jax.ShapeDtypeStruct((N, 2 * H), _F32),
        ],
    )(h, Wd, Ws)


def _msg_body(g_ref, ea_ref, we_ref, b_ref, par_ref, o_ref):
    zf = g_ref[...] + jnp.dot(ea_ref[...], we_ref[...],
                              preferred_element_type=_F32) + b_ref[...]
    zg = zf[:, :H]
    zc = zf[:, H:]
    sig = 1.0 / (1.0 + jnp.exp(-zg))
    sp = jnp.maximum(zc, 0.0) + jnp.log(1.0 + jnp.exp(-jnp.abs(zc)))
    m = sig * sp
    par = par_ref[...]
    o_ref[...] = jnp.concatenate([m * (1.0 - par), m * par], axis=1)


def _make_msg(ne):
    def call(G, ea, We, b2, par):
        return pl.pallas_call(
            _msg_body,
            grid=(ne // BE,),
            in_specs=[
                pl.BlockSpec((BE, 2 * H), lambda i: (i, 0)),
                pl.BlockSpec((BE, 16), lambda i: (i, 0)),
                pl.BlockSpec((16, 2 * H), lambda i: (0, 0)),
                pl.BlockSpec((1, 2 * H), lambda i: (0, 0)),
                pl.BlockSpec((BE, 1), lambda i: (i, 0)),
            ],
            out_specs=pl.BlockSpec((BE, 2 * H), lambda i: (i, 0)),
            out_shape=jax.ShapeDtypeStruct((ne, 2 * H), _F32),
        )(G, ea, We, b2, par)
    return call


def _upd_body(h_ref, a_ref, b_ref, o_ref):
    t = h_ref[...] + a_ref[...] + b_ref[...]
    o_ref[...] = jnp.maximum(t, 0.0) + jnp.log(1.0 + jnp.exp(-jnp.abs(t)))


def _upd(h, aggA, aggB):
    return pl.pallas_call(
        _upd_body,
        grid=(N // BN,),
        in_specs=[
            pl.BlockSpec((BN, H), lambda i: (i, 0)),
            pl.BlockSpec((BN, H), lambda i: (i, 0)),
            pl.BlockSpec((BN, H), lambda i: (i, 0)),
        ],
        out_specs=pl.BlockSpec((BN, H), lambda i: (i, 0)),
        out_shape=jax.ShapeDtypeStruct((N, H), _F32),
    )(h, aggA, aggB)


def _make_gather(ne):
    epw = ne // _NW
    nfull = epw // _GCH
    grem = epw - nfull * _GCH
    npair = (nfull - 1) // 2
    k0 = 2 * npair

    ngrp = max((nfull - 5) // 3, 0)

    @functools.partial(
        pl.kernel,
        out_type=jax.ShapeDtypeStruct((ne, 2 * H), _F32),
        mesh=_sc_mesh,
        scratch_types=[
            pltpu.VMEM((epw,), jnp.int32),
            pltpu.VMEM((epw,), jnp.int32),
            pltpu.VMEM((_GCH, 2 * H), _F32),
            pltpu.VMEM((_GCH, 2 * H), _F32),
            pltpu.VMEM((_GCH, 2 * H), _F32),
            pltpu.SemaphoreType.DMA,
            pltpu.SemaphoreType.DMA,
            pltpu.SemaphoreType.DMA,
            pltpu.SemaphoreType.DMA,
            pltpu.SemaphoreType.DMA,
            pltpu.SemaphoreType.DMA,
            pltpu.SemaphoreType.DMA,
            pltpu.SemaphoreType.DMA,
            pltpu.SemaphoreType.DMA,
        ],
    )
    def gather_k(pd_hbm, ps_hbm, dst_hbm, src_hbm, out_hbm, dsti, srci,
                 buf0, buf1, buf2, semA0, semA1, semA2,
                 semB0, semB1, semB2, semS0, semS1, semS2):
        wid = lax.axis_index("s") * _NC + lax.axis_index("c")
        base = wid * epw
        pltpu.sync_copy(dst_hbm.at[pl.ds(base, epw)], dsti)
        pltpu.sync_copy(src_hbm.at[pl.ds(base, epw)], srci)

        bufs = (buf0, buf1, buf2)
        semsA = (semA0, semA1, semA2)
        semsB = (semB0, semB1, semB2)
        semsS = (semS0, semS1, semS2)

        def issue_a(off, slot):
            pltpu.async_copy(pd_hbm.at[dsti.at[pl.ds(off, _GCH)]],
                             bufs[slot], semsA[slot])

        def wait_a(off, slot):
            pltpu.make_async_copy(pd_hbm.at[dsti.at[pl.ds(off, _GCH)]],
                                  bufs[slot], semsA[slot]).wait()

        def issue_b(off, slot):
            pltpu.async_copy(ps_hbm.at[srci.at[pl.ds(off, _GCH)]],
                             bufs[slot], semsB[slot], add=True)

        def wait_b(off, slot):
            pltpu.make_async_copy(ps_hbm.at[srci.at[pl.ds(off, _GCH)]],
                                  bufs[slot], semsB[slot]).wait()

        def issue_s(off, slot):
            pltpu.async_copy(bufs[slot], out_hbm.at[pl.ds(base + off, _GCH)],
                             semsS[slot])

        def wait_s(off, slot):
            pltpu.make_async_copy(bufs[slot],
                                  out_hbm.at[pl.ds(base + off, _GCH)],
                                  semsS[slot]).wait()

        def step(k_off, u):
            if u is None:
                return
            slot = u % 3
            nslot = (u + 2) % 3
            pltpu.make_async_copy(
                bufs[nslot],
                out_hbm.at[pl.ds(base + k_off - _GCH, _GCH)],
                semsS[nslot]).wait()
            issue_a(k_off + 2 * _GCH, nslot)
            wait_a(k_off, slot)
            issue_b(k_off, slot)
            wait_b(k_off, slot)
            issue_s(k_off, slot)

        issue_a(0, 0)
        issue_a(_GCH, 1)
        issue_a(2 * _GCH, 2)
        wait_a(0, 0)
        issue_b(0, 0)
        wait_b(0, 0)
        issue_s(0, 0)

        @pl.loop(0, ngrp)
        def _(j):
            k = (3 * j + 1) * _GCH
            step(k, 1)
            step(k + _GCH, 2)
            step(k + 2 * _GCH, 0)

        issued = 3 * ngrp + 3 if ngrp else 3
        for k in range(3 * ngrp + 1, nfull):
            slot = k % 3
            nslot = (k + 2) % 3
            wait_s((k - 1) * _GCH, nslot)
            if k + 2 < nfull and k + 2 >= issued:
                issue_a((k + 2) * _GCH, nslot)
                issued = k + 3
            wait_a(k * _GCH, slot)
            issue_b(k * _GCH, slot)
            wait_b(k * _GCH, slot)
            issue_s(k * _GCH, slot)
        if grem:
            roff = nfull * _GCH
            slot = nfull % 3
            pltpu.async_copy(pd_hbm.at[dsti.at[pl.ds(roff, grem)]],
                             bufs[slot].at[pl.ds(0, grem)], semsA[slot]).wait()
            pltpu.async_copy(ps_hbm.at[srci.at[pl.ds(roff, grem)]],
                             bufs[slot].at[pl.ds(0, grem)], semsB[slot],
                             add=True).wait()
            pltpu.sync_copy(bufs[slot].at[pl.ds(0, grem)],
                            out_hbm.at[pl.ds(base + roff, grem)])
        wait_s((nfull - 1) * _GCH, (nfull - 1) % 3)

    return gather_k


def _make_scatter(ne):
    etp = ne // _NS
    sfull = etp // _SCH
    srem = etp - sfull * _SCH
    npair = (sfull - 1) // 2
    k0 = 2 * npair

    @functools.partial(
        pl.kernel,
        out_type=jax.ShapeDtypeStruct((2 * _RSH, 2 * H), _F32),
        mesh=_sc_mesh,
        scratch_types=[
            pltpu.VMEM((_SCH,), jnp.int32),
            pltpu.VMEM((_SCH,), jnp.int32),
            pltpu.VMEM((_SCH, 2 * H), _F32),
            pltpu.VMEM((_SCH, 2 * H), _F32),
            pltpu.VMEM((_SCH,), jnp.int32),
            pltpu.VMEM_SHARED((_RSH, 2 * H), _F32),
            pltpu.SemaphoreType.DMA,
            pltpu.SemaphoreType.DMA,
            pltpu.SemaphoreType.DMA,
            pltpu.SemaphoreType.DMA,
        ],
    )
    def scatter_k(msg_hbm, dst_hbm, zeros_hbm, out_hbm,
                  idx0, idx1, mbuf0, mbuf1, sidx, aggsh,
                  semM0, semM1, semI0, semI1):
        c = lax.axis_index("c")
        s = lax.axis_index("s")
        nbase = c * _NHALF
        obase = c * _RSH
        ebase = s * etp
        pltpu.sync_copy(zeros_hbm, mbuf0)
        for z in range(_ZR // _SCH):
            pltpu.sync_copy(mbuf0, aggsh.at[pl.ds(s * _ZR + z * _SCH, _SCH)])
        pltpu.sync_copy(mbuf0.at[pl.ds(0, _ZR - (_ZR // _SCH) * _SCH)],
                        aggsh.at[pl.ds(s * _ZR + (_ZR // _SCH) * _SCH,
                                       _ZR - (_ZR // _SCH) * _SCH)])
        plsc.subcore_barrier()

        mbufs = (mbuf0, mbuf1)
        idxs = (idx0, idx1)
        semsM = (semM0, semM1)
        semsI = (semI0, semI1)

        def issue_l(k, slot):
            goff = ebase + k * _SCH
            pltpu.async_copy(msg_hbm.at[pl.ds(goff, _SCH)], mbufs[slot], semsM[slot])
            pltpu.async_copy(dst_hbm.at[pl.ds(goff, _SCH)], idxs[slot], semsI[slot])

        def wait_l(k, slot):
            goff = ebase + k * _SCH
            pltpu.make_async_copy(msg_hbm.at[pl.ds(goff, _SCH)],
                                  mbufs[slot], semsM[slot]).wait()
            pltpu.make_async_copy(dst_hbm.at[pl.ds(goff, _SCH)],
                                  idxs[slot], semsI[slot]).wait()

        def finish(slot, nvec):
            for v in range(nvec):
                d = idxs[slot][pl.ds(v * 16, 16)]
                li = d - nbase
                ok = (li >= 0) & (li < _NHALF)
                sidx[pl.ds(v * 16, 16)] = jnp.where(ok, li >> 1, _DUMMY)
            for v in range(nvec, _SCH // 16):
                sidx[pl.ds(v * 16, 16)] = jnp.full((16,), _DUMMY, jnp.int32)
            pltpu.sync_copy(mbufs[slot], aggsh.at[sidx], add=True)

        issue_l(0, 0)

        @pl.loop(0, npair)
        def _(j):
            a = 2 * j
            issue_l(a + 1, 1)
            wait_l(a, 0)
            finish(0, _SCH // 16)
            issue_l(a + 2, 0)
            wait_l(a + 1, 1)
            finish(1, _SCH // 16)

        wait_l(k0, 0)
        finish(0, _SCH // 16)
        if k0 + 1 < sfull:
            issue_l(k0 + 1, 1)
            wait_l(k0 + 1, 1)
            finish(1, _SCH // 16)
        if srem:
            roff = ebase + sfull * _SCH
            pltpu.async_copy(msg_hbm.at[pl.ds(roff, srem)],
                             mbuf0.at[pl.ds(0, srem)], semM0).wait()
            pltpu.async_copy(dst_hbm.at[pl.ds(roff, srem)],
                             idx0.at[pl.ds(0, srem)], semI0).wait()
            for v in range(srem // 16):
                d = idx0[pl.ds(v * 16, 16)]
                li = d - nbase
                ok = (li >= 0) & (li < _NHALF)
                sidx[pl.ds(v * 16, 16)] = jnp.where(ok, li >> 1, _DUMMY)
            for v in range(srem // 16, _SCH // 16):
                sidx[pl.ds(v * 16, 16)] = jnp.full((16,), _DUMMY, jnp.int32)
            pltpu.sync_copy(mbuf0, aggsh.at[sidx], add=True)
        plsc.subcore_barrier()

        def out_rows(roff2, rsz):
            pltpu.sync_copy(aggsh.at[pl.ds(roff2, rsz)], mbuf0.at[pl.ds(0, rsz)])
            pltpu.sync_copy(mbuf0.at[pl.ds(0, rsz)],
                            out_hbm.at[pl.ds(obase + roff2, rsz)])

        @pl.when(s < _NS - 1)
        def _():
            for z in range(_OROWS // _SCH):
                out_rows(s * _OROWS + z * _SCH, _SCH)
            out_rows(s * _OROWS + (_OROWS // _SCH) * _SCH,
                     _OROWS - (_OROWS // _SCH) * _SCH)

        @pl.when(s == _NS - 1)
        def _():
            last = 744
            for z in range(last // _SCH):
                out_rows((_NS - 1) * _OROWS + z * _SCH, _SCH)
            out_rows((_NS - 1) * _OROWS + (last // _SCH) * _SCH,
                     last - (last // _SCH) * _SCH)

    return scatter_k


_gathers = tuple(_make_gather(ne) for ne in _SPLITS)
_scatters = tuple(_make_scatter(ne) for ne in _SPLITS)
_msgs = tuple(_make_msg(ne) for ne in _SPLITS)


def _upd_body3(h_ref, a_ref, b_ref, c_ref, o_ref):
    t = h_ref[...] + a_ref[...] + b_ref[...] + c_ref[...]
    o_ref[...] = jnp.maximum(t, 0.0) + jnp.log(1.0 + jnp.exp(-jnp.abs(t)))


def _upd3(h, aggA, aggB, aggC):
    return pl.pallas_call(
        _upd_body3,
        grid=(N // BN,),
        in_specs=[
            pl.BlockSpec((BN, H), lambda i: (i, 0)),
            pl.BlockSpec((BN, H), lambda i: (i, 0)),
            pl.BlockSpec((BN, H), lambda i: (i, 0)),
            pl.BlockSpec((BN, H), lambda i: (i, 0)),
        ],
        out_specs=pl.BlockSpec((BN, H), lambda i: (i, 0)),
        out_shape=jax.ShapeDtypeStruct((N, H), _F32),
    )(h, aggA, aggB, aggC)


def kernel(x, edge_index, edge_attr, W_embed, b_embed,
           W_full_0, b_full_0, W_full_1, b_full_1, W_full_2, b_full_2):
    src = edge_index[0]
    dst = edge_index[1]
    xp = jnp.pad(x, ((0, 0), (0, 128 - x.shape[1])))
    Wp = jnp.pad(W_embed, ((0, 128 - W_embed.shape[0]), (0, 0)))
    h = _embed(xp, Wp, b_embed.reshape(1, H))
    zeros_sh = jnp.zeros((_SCH, 2 * H), _F32)
    par = (dst & 1).astype(_F32).reshape(E, 1)
    offs = [0]
    for ne in _SPLITS:
        offs.append(offs[-1] + ne)
    dsts = [dst[offs[i]:offs[i + 1]] for i in range(len(_SPLITS))]
    srcs = [src[offs[i]:offs[i + 1]] for i in range(len(_SPLITS))]
    eas = [edge_attr[offs[i]:offs[i + 1]] for i in range(len(_SPLITS))]
    pars = [par[offs[i]:offs[i + 1]] for i in range(len(_SPLITS))]
    for W, b in ((W_full_0, b_full_0), (W_full_1, b_full_1), (W_full_2, b_full_2)):
        Wd, Ws, We = W[:H], W[H:2 * H], W[2 * H:]
        b2 = b.reshape(1, 2 * H)
        Pd, Ps = _proj(h, Wd, Ws)
        Gs = [_gathers[i](Pd, Ps, dsts[i], srcs[i]) for i in range(len(_SPLITS))]
        Ms = [_msgs[i](Gs[i], eas[i], We, b2, pars[i]) for i in range(len(_SPLITS))]
        aps = [_scatters[i](Ms[i], dsts[i], zeros_sh) for i in range(len(_SPLITS))]
        aggs = [jnp.concatenate([ap[:_PROWS], ap[_RSH:_RSH + _PROWS]],
                                axis=0).reshape(N, H) for ap in aps]
        h = _upd3(h, aggs[0], aggs[1], aggs[2])
    return h


# speedup vs baseline: 1.0815x; 1.0148x over previous
import functools

import jax
import jax.numpy as jnp
from jax import lax
from jax.experimental import pallas as pl
from jax.experimental.pallas import tpu as pltpu
from jax.experimental.pallas import tpu_sc as plsc

N = 50000
E = 800000
H = 64
BN = 2000
BE = 1600

_F32 = jnp.float32

_NC = 2
_NS = 16
_NW = _NC * _NS
_GCH = 128

_NHALF = N // _NC
_PROWS = _NHALF // 2
_DUMMY = _PROWS
_RSH = 12544
_ZR = _RSH // _NS
_SCH = 96
_OROWS = 784

_SPLITS = (204800, 204800, 204800, 185600)

_sc_mesh = plsc.VectorSubcoreMesh(core_axis_name="c", subcore_axis_name="s")


def _embed_body(x_ref, w_ref, b_ref, o_ref):
    o_ref[...] = (
        jnp.dot(x_ref[...], w_ref[...], preferred_element_type=_F32) + b_ref[...]
    )


def _embed(xp, Wp, b2):
    return pl.pallas_call(
        _embed_body,
        grid=(N // BN,),
        in_specs=[
            pl.BlockSpec((BN, 128), lambda i: (i, 0)),
            pl.BlockSpec((128, H), lambda i: (0, 0)),
            pl.BlockSpec((1, H), lambda i: (0, 0)),
        ],
        out_specs=pl.BlockSpec((BN, H), lambda i: (i, 0)),
        out_shape=jax.ShapeDtypeStruct((N, H), _F32),
    )(xp, Wp, b2)


def _proj_body(h_ref, wd_ref, ws_ref, pd_ref, ps_ref):
    h = h_ref[...]
    pd_ref[...] = jnp.dot(h, wd_ref[...], preferred_element_type=_F32)
    ps_ref[...] = jnp.dot(h, ws_ref[...], preferred_element_type=_F32)


def _proj(h, Wd, Ws):
    return pl.pallas_call(
        _proj_body,
        grid=(N // BN,),
        in_specs=[
            pl.BlockSpec((BN, H), lambda i: (i, 0)),
            pl.BlockSpec((H, 2 * H), lambda i: (0, 0)),
            pl.BlockSpec((H, 2 * H), lambda i: (0, 0)),
        ],
        out_specs=[
            pl.BlockSpec((BN, 2 * H), lambda i: (i, 0)),
            pl.BlockSpec((BN, 2 * H), lambda i: (i, 0)),
        ],
        out_shape=[
            jax.ShapeDtypeStruct((N, 2 * H), _F32),
            jax.ShapeDtypeStruct((N, 2 * H), _F32),
        ],
    )(h, Wd, Ws)


def _msg_body(g_ref, ea_ref, we_ref, b_ref, par_ref, o_ref):
    zf = g_ref[...] + jnp.dot(ea_ref[...], we_ref[...],
                              preferred_element_type=_F32) + b_ref[...]
    zg = zf[:, :H]
    zc = zf[:, H:]
    sig = 1.0 / (1.0 + jnp.exp(-zg))
    sp = jnp.maximum(zc, 0.0) + jnp.log(1.0 + jnp.exp(-jnp.abs(zc)))
    m = sig * sp
    par = par_ref[...]
    o_ref[...] = jnp.concatenate([m * (1.0 - par), m * par], axis=1)


def _make_msg(ne):
    def call(G, ea, We, b2, par):
        return pl.pallas_call(
            _msg_body,
            grid=(ne // BE,),
            in_specs=[
                pl.BlockSpec((BE, 2 * H), lambda i: (i, 0)),
                pl.BlockSpec((BE, 16), lambda i: (i, 0)),
                pl.BlockSpec((16, 2 * H), lambda i: (0, 0)),
                pl.BlockSpec((1, 2 * H), lambda i: (0, 0)),
                pl.BlockSpec((BE, 1), lambda i: (i, 0)),
            ],
            out_specs=pl.BlockSpec((BE, 2 * H), lambda i: (i, 0)),
            out_shape=jax.ShapeDtypeStruct((ne, 2 * H), _F32),
        )(G, ea, We, b2, par)
    return call


def _upd_body(h_ref, a_ref, b_ref, o_ref):
    t = h_ref[...] + a_ref[...] + b_ref[...]
    o_ref[...] = jnp.maximum(t, 0.0) + jnp.log(1.0 + jnp.exp(-jnp.abs(t)))


def _upd(h, aggA, aggB):
    return pl.pallas_call(
        _upd_body,
        grid=(N // BN,),
        in_specs=[
            pl.BlockSpec((BN, H), lambda i: (i, 0)),
            pl.BlockSpec((BN, H), lambda i: (i, 0)),
            pl.BlockSpec((BN, H), lambda i: (i, 0)),
        ],
        out_specs=pl.BlockSpec((BN, H), lambda i: (i, 0)),
        out_shape=jax.ShapeDtypeStruct((N, H), _F32),
    )(h, aggA, aggB)


def _make_gather(ne):
    epw = ne // _NW
    nfull = epw // _GCH
    grem = epw - nfull * _GCH
    npair = (nfull - 1) // 2
    k0 = 2 * npair

    ngrp = max((nfull - 5) // 3, 0)

    @functools.partial(
        pl.kernel,
        out_type=jax.ShapeDtypeStruct((ne, 2 * H), _F32),
        mesh=_sc_mesh,
        scratch_types=[
            pltpu.VMEM((epw,), jnp.int32),
            pltpu.VMEM((epw,), jnp.int32),
            pltpu.VMEM((_GCH, 2 * H), _F32),
            pltpu.VMEM((_GCH, 2 * H), _F32),
            pltpu.VMEM((_GCH, 2 * H), _F32),
            pltpu.SemaphoreType.DMA,
            pltpu.SemaphoreType.DMA,
            pltpu.SemaphoreType.DMA,
            pltpu.SemaphoreType.DMA,
            pltpu.SemaphoreType.DMA,
            pltpu.SemaphoreType.DMA,
            pltpu.SemaphoreType.DMA,
            pltpu.SemaphoreType.DMA,
            pltpu.SemaphoreType.DMA,
        ],
    )
    def gather_k(pd_hbm, ps_hbm, dst_hbm, src_hbm, out_hbm, dsti, srci,
                 buf0, buf1, buf2, semA0, semA1, semA2,
                 semB0, semB1, semB2, semS0, semS1, semS2):
        wid = lax.axis_index("s") * _NC + lax.axis_index("c")
        base = wid * epw
        pltpu.sync_copy(dst_hbm.at[pl.ds(base, epw)], dsti)
        pltpu.sync_copy(src_hbm.at[pl.ds(base, epw)], srci)

        bufs = (buf0, buf1, buf2)
        semsA = (semA0, semA1, semA2)
        semsB = (semB0, semB1, semB2)
        semsS = (semS0, semS1, semS2)

        def issue_a(off, slot):
            pltpu.async_copy(pd_hbm.at[dsti.at[pl.ds(off, _GCH)]],
                             bufs[slot], semsA[slot])

        def wait_a(off, slot):
            pltpu.make_async_copy(pd_hbm.at[dsti.at[pl.ds(off, _GCH)]],
                                  bufs[slot], semsA[slot]).wait()

        def issue_b(off, slot):
            pltpu.async_copy(ps_hbm.at[srci.at[pl.ds(off, _GCH)]],
                             bufs[slot], semsB[slot], add=True)

        def wait_b(off, slot):
            pltpu.make_async_copy(ps_hbm.at[srci.at[pl.ds(off, _GCH)]],
                                  bufs[slot], semsB[slot]).wait()

        def issue_s(off, slot):
            pltpu.async_copy(bufs[slot], out_hbm.at[pl.ds(base + off, _GCH)],
                             semsS[slot])

        def wait_s(off, slot):
            pltpu.make_async_copy(bufs[slot],
                                  out_hbm.at[pl.ds(base + off, _GCH)],
                                  semsS[slot]).wait()

        def step(k_off, u):
            if u is None:
                return
            slot = u % 3
            nslot = (u + 2) % 3
            pltpu.make_async_copy(
                bufs[nslot],
                out_hbm.at[pl.ds(base + k_off - _GCH, _GCH)],
                semsS[nslot]).wait()
            issue_a(k_off + 2 * _GCH, nslot)
            wait_a(k_off, slot)
            issue_b(k_off, slot)
            wait_b(k_off, slot)
            issue_s(k_off, slot)

        issue_a(0, 0)
        issue_a(_GCH, 1)
        issue_a(2 * _GCH, 2)
        wait_a(0, 0)
        issue_b(0, 0)
        wait_b(0, 0)
        issue_s(0, 0)

        @pl.loop(0, ngrp)
        def _(j):
            k = (3 * j + 1) * _GCH
            step(k, 1)
            step(k + _GCH, 2)
            step(k + 2 * _GCH, 0)

        issued = 3 * ngrp + 3 if ngrp else 3
        for k in range(3 * ngrp + 1, nfull):
            slot = k % 3
            nslot = (k + 2) % 3
            wait_s((k - 1) * _GCH, nslot)
            if k + 2 < nfull and k + 2 >= issued:
                issue_a((k + 2) * _GCH, nslot)
                issued = k + 3
            wait_a(k * _GCH, slot)
            issue_b(k * _GCH, slot)
            wait_b(k * _GCH, slot)
            issue_s(k * _GCH, slot)
        if grem:
            roff = nfull * _GCH
            slot = nfull % 3
            pltpu.async_copy(pd_hbm.at[dsti.at[pl.ds(roff, grem)]],
                             bufs[slot].at[pl.ds(0, grem)], semsA[slot]).wait()
            pltpu.async_copy(ps_hbm.at[srci.at[pl.ds(roff, grem)]],
                             bufs[slot].at[pl.ds(0, grem)], semsB[slot],
                             add=True).wait()
            pltpu.sync_copy(bufs[slot].at[pl.ds(0, grem)],
                            out_hbm.at[pl.ds(base + roff, grem)])
        wait_s((nfull - 1) * _GCH, (nfull - 1) % 3)

    return gather_k


def _make_scatter(ne):
    etp = ne // _NS
    sfull = etp // _SCH
    srem = etp - sfull * _SCH
    npair = (sfull - 1) // 2
    k0 = 2 * npair

    @functools.partial(
        pl.kernel,
        out_type=jax.ShapeDtypeStruct((2 * _RSH, 2 * H), _F32),
        mesh=_sc_mesh,
        scratch_types=[
            pltpu.VMEM((_SCH,), jnp.int32),
            pltpu.VMEM((_SCH,), jnp.int32),
            pltpu.VMEM((_SCH, 2 * H), _F32),
            pltpu.VMEM((_SCH, 2 * H), _F32),
            pltpu.VMEM((_SCH,), jnp.int32),
            pltpu.VMEM_SHARED((_RSH, 2 * H), _F32),
            pltpu.SemaphoreType.DMA,
            pltpu.SemaphoreType.DMA,
            pltpu.SemaphoreType.DMA,
            pltpu.SemaphoreType.DMA,
        ],
    )
    def scatter_k(msg_hbm, dst_hbm, zeros_hbm, out_hbm,
                  idx0, idx1, mbuf0, mbuf1, sidx, aggsh,
                  semM0, semM1, semI0, semI1):
        c = lax.axis_index("c")
        s = lax.axis_index("s")
        nbase = c * _NHALF
        obase = c * _RSH
        ebase = s * etp
        pltpu.sync_copy(zeros_hbm, mbuf0)
        for z in range(_ZR // _SCH):
            pltpu.sync_copy(mbuf0, aggsh.at[pl.ds(s * _ZR + z * _SCH, _SCH)])
        pltpu.sync_copy(mbuf0.at[pl.ds(0, _ZR - (_ZR // _SCH) * _SCH)],
                        aggsh.at[pl.ds(s * _ZR + (_ZR // _SCH) * _SCH,
                                       _ZR - (_ZR // _SCH) * _SCH)])
        plsc.subcore_barrier()

        mbufs = (mbuf0, mbuf1)
        idxs = (idx0, idx1)
        semsM = (semM0, semM1)
        semsI = (semI0, semI1)

        def issue_l(k, slot):
            goff = ebase + k * _SCH
            pltpu.async_copy(msg_hbm.at[pl.ds(goff, _SCH)], mbufs[slot], semsM[slot])
            pltpu.async_copy(dst_hbm.at[pl.ds(goff, _SCH)], idxs[slot], semsI[slot])

        def wait_l(k, slot):
            goff = ebase + k * _SCH
            pltpu.make_async_copy(msg_hbm.at[pl.ds(goff, _SCH)],
                                  mbufs[slot], semsM[slot]).wait()
            pltpu.make_async_copy(dst_hbm.at[pl.ds(goff, _SCH)],
                                  idxs[slot], semsI[slot]).wait()

        def finish(slot, nvec):
            for v in range(nvec):
                d = idxs[slot][pl.ds(v * 16, 16)]
                li = d - nbase
                ok = (li >= 0) & (li < _NHALF)
                sidx[pl.ds(v * 16, 16)] = jnp.where(ok, li >> 1, _DUMMY)
            for v in range(nvec, _SCH // 16):
                sidx[pl.ds(v * 16, 16)] = jnp.full((16,), _DUMMY, jnp.int32)
            pltpu.sync_copy(mbufs[slot], aggsh.at[sidx], add=True)

        issue_l(0, 0)

        @pl.loop(0, npair)
        def _(j):
            a = 2 * j
            issue_l(a + 1, 1)
            wait_l(a, 0)
            finish(0, _SCH // 16)
            issue_l(a + 2, 0)
            wait_l(a + 1, 1)
            finish(1, _SCH // 16)

        wait_l(k0, 0)
        finish(0, _SCH // 16)
        if k0 + 1 < sfull:
            issue_l(k0 + 1, 1)
            wait_l(k0 + 1, 1)
            finish(1, _SCH // 16)
        if srem:
            roff = ebase + sfull * _SCH
            pltpu.async_copy(msg_hbm.at[pl.ds(roff, srem)],
                             mbuf0.at[pl.ds(0, srem)], semM0).wait()
            pltpu.async_copy(dst_hbm.at[pl.ds(roff, srem)],
                             idx0.at[pl.ds(0, srem)], semI0).wait()
            for v in range(srem // 16):
                d = idx0[pl.ds(v * 16, 16)]
                li = d - nbase
                ok = (li >= 0) & (li < _NHALF)
                sidx[pl.ds(v * 16, 16)] = jnp.where(ok, li >> 1, _DUMMY)
            for v in range(srem // 16, _SCH // 16):
                sidx[pl.ds(v * 16, 16)] = jnp.full((16,), _DUMMY, jnp.int32)
            pltpu.sync_copy(mbuf0, aggsh.at[sidx], add=True)
        plsc.subcore_barrier()

        def out_rows(roff2, rsz):
            pltpu.sync_copy(aggsh.at[pl.ds(roff2, rsz)], mbuf0.at[pl.ds(0, rsz)])
            pltpu.sync_copy(mbuf0.at[pl.ds(0, rsz)],
                            out_hbm.at[pl.ds(obase + roff2, rsz)])

        @pl.when(s < _NS - 1)
        def _():
            for z in range(_OROWS // _SCH):
                out_rows(s * _OROWS + z * _SCH, _SCH)
            out_rows(s * _OROWS + (_OROWS // _SCH) * _SCH,
                     _OROWS - (_OROWS // _SCH) * _SCH)

        @pl.when(s == _NS - 1)
        def _():
            last = 744
            for z in range(last // _SCH):
                out_rows((_NS - 1) * _OROWS + z * _SCH, _SCH)
            out_rows((_NS - 1) * _OROWS + (last // _SCH) * _SCH,
                     last - (last // _SCH) * _SCH)

    return scatter_k


_gathers = tuple(_make_gather(ne) for ne in _SPLITS)
_scatters = tuple(_make_scatter(ne) for ne in _SPLITS)
_msgs = tuple(_make_msg(ne) for ne in _SPLITS)


def _upd_bodyN(*refs):
    o_ref = refs[-1]
    t = refs[0][...]
    for r in refs[1:-1]:
        t = t + r[...]
    o_ref[...] = jnp.maximum(t, 0.0) + jnp.log(1.0 + jnp.exp(-jnp.abs(t)))


def _updN(h, *aggs):
    n_in = 1 + len(aggs)
    return pl.pallas_call(
        _upd_bodyN,
        grid=(N // BN,),
        in_specs=[pl.BlockSpec((BN, H), lambda i: (i, 0)) for _ in range(n_in)],
        out_specs=pl.BlockSpec((BN, H), lambda i: (i, 0)),
        out_shape=jax.ShapeDtypeStruct((N, H), _F32),
    )(h, *aggs)


def kernel(x, edge_index, edge_attr, W_embed, b_embed,
           W_full_0, b_full_0, W_full_1, b_full_1, W_full_2, b_full_2):
    src = edge_index[0]
    dst = edge_index[1]
    xp = jnp.pad(x, ((0, 0), (0, 128 - x.shape[1])))
    Wp = jnp.pad(W_embed, ((0, 128 - W_embed.shape[0]), (0, 0)))
    h = _embed(xp, Wp, b_embed.reshape(1, H))
    zeros_sh = jnp.zeros((_SCH, 2 * H), _F32)
    par = (dst & 1).astype(_F32).reshape(E, 1)
    offs = [0]
    for ne in _SPLITS:
        offs.append(offs[-1] + ne)
    dsts = [dst[offs[i]:offs[i + 1]] for i in range(len(_SPLITS))]
    srcs = [src[offs[i]:offs[i + 1]] for i in range(len(_SPLITS))]
    eas = [edge_attr[offs[i]:offs[i + 1]] for i in range(len(_SPLITS))]
    pars = [par[offs[i]:offs[i + 1]] for i in range(len(_SPLITS))]
    for W, b in ((W_full_0, b_full_0), (W_full_1, b_full_1), (W_full_2, b_full_2)):
        Wd, Ws, We = W[:H], W[H:2 * H], W[2 * H:]
        b2 = b.reshape(1, 2 * H)
        Pd, Ps = _proj(h, Wd, Ws)
        Gs = [_gathers[i](Pd, Ps, dsts[i], srcs[i]) for i in range(len(_SPLITS))]
        Ms = [_msgs[i](Gs[i], eas[i], We, b2, pars[i]) for i in range(len(_SPLITS))]
        aps = [_scatters[i](Ms[i], dsts[i], zeros_sh) for i in range(len(_SPLITS))]
        aggs = [jnp.concatenate([ap[:_PROWS], ap[_RSH:_RSH + _PROWS]],
                                axis=0).reshape(N, H) for ap in aps]
        h = _updN(h, *aggs)
    return h
